# baseline (speedup 1.0000x reference)
import functools

import jax
import jax.numpy as jnp
import numpy as np
from jax import lax
from jax.experimental import pallas as pl
from jax.experimental.pallas import tpu as pltpu
from jax.experimental.pallas import tpu_sc as plsc

TIME_RES = 8
NL = 8
F = 4
H = 1 << 14
NB = 4
N_PTS = 1048576
_PLS = float(np.exp2(np.log2(32768 / 512) / (NL - 1)))
SCALES = np.array(
    [np.exp2(l * np.log2(_PLS)) * 512 - 1.0 for l in range(NL)], dtype=np.float32
)
PRIME1 = np.uint32(2654435761)
HMASK = np.uint32(H - 1)

NC = 2
NS = 16
LANES = 16
NW = NC * NS

NCHUNK = NW // NL
CHUNK_PTS = N_PTS // NCHUNK
PB = 8192
NKB = CHUNK_PTS // PB
CH = 2048
CROW = 64


def _sc_body(s1_hbm, s2_hbm, const_hbm, x_hbm, out_hbm,
             comb_v, s1buf, s2buf, pbuf, cbuf, xybuf, obuf):
    cid = lax.axis_index("c")
    sid = lax.axis_index("s")
    wid = sid * NC + cid
    level = wid % NL
    chunk = wid // NL

    pltpu.sync_copy(const_hbm.at[pl.ds(wid * CROW, CROW)], cbuf)
    pat1 = cbuf[pl.ds(0, LANES)]
    pat2 = cbuf[pl.ds(LANES, LANES)]
    scale = cbuf[pl.ds(2 * LANES, LANES)]
    iota = lax.iota(jnp.int32, LANES)

    lhf = level * (H * F)

    def stage_a(ci, carry):
        off = lhf + ci * (CH * F)
        pltpu.sync_copy(s1_hbm.at[pl.ds(off, CH * F)], s1buf)
        pltpu.sync_copy(s2_hbm.at[pl.ds(off, CH * F)], s2buf)

        def premul(g, c_):
            s = pl.ds(g * LANES, LANES)
            pbuf[s] = s1buf[s] * pat1 + s2buf[s] * pat2
            return c_

        lax.fori_loop(0, CH * F // LANES, premul, carry)

        def reduce4(g, c_):
            idx = g * (LANES * F) + iota * F
            acc = plsc.load_gather(pbuf, [idx])
            acc = acc + plsc.load_gather(pbuf, [idx + 1])
            acc = acc + plsc.load_gather(pbuf, [idx + 2])
            acc = acc + plsc.load_gather(pbuf, [idx + 3])
            comb_v[pl.ds(ci * CH + g * LANES, LANES)] = acc
            return c_

        return lax.fori_loop(0, CH // LANES, reduce4, carry)

    lax.fori_loop(0, H // CH, stage_a, 0)

    pbase = chunk * CHUNK_PTS

    def stage_b(k, carry):
        xoff = pbase + k * PB
        pltpu.sync_copy(x_hbm.at[pl.ds(2 * xoff, 2 * PB)], xybuf)

        def inner(g, c_):
            xs = plsc.load_gather(xybuf, [g * (2 * LANES) + iota * 2])
            ys = plsc.load_gather(xybuf, [g * (2 * LANES) + iota * 2 + 1])
            px = xs * scale + 0.5
            py = ys * scale + 0.5
            ix = px.astype(jnp.int32)
            iy = py.astype(jnp.int32)
            wx = px - ix.astype(jnp.float32)
            wy = py - iy.astype(jnp.float32)
            ux = ix.astype(jnp.uint32)
            uy = iy.astype(jnp.uint32)
            hy0 = uy * PRIME1
            hy1 = hy0 + PRIME1
            ux1 = ux + np.uint32(1)
            h00 = ((ux ^ hy0) & HMASK).astype(jnp.int32)
            h10 = ((ux1 ^ hy0) & HMASK).astype(jnp.int32)
            h01 = ((ux ^ hy1) & HMASK).astype(jnp.int32)
            h11 = ((ux1 ^ hy1) & HMASK).astype(jnp.int32)
            g00 = plsc.load_gather(comb_v, [h00])
            g10 = plsc.load_gather(comb_v, [h10])
            g01 = plsc.load_gather(comb_v, [h01])
            g11 = plsc.load_gather(comb_v, [h11])
            gx0 = g00 + (g10 - g00) * wx
            gx1 = g01 + (g11 - g01) * wx
            res = gx0 + (gx1 - gx0) * wy
            obuf[pl.ds(g * LANES, LANES)] = res
            return c_

        lax.fori_loop(0, PB // LANES, inner, carry)
        pltpu.sync_copy(obuf, out_hbm.at[pl.ds(level * N_PTS + xoff, PB)])
        return carry

    lax.fori_loop(0, NKB, stage_b, 0)


_sc_call = functools.partial(
    pl.kernel,
    out_type=jax.ShapeDtypeStruct((NL * N_PTS,), jnp.float32),
    mesh=plsc.VectorSubcoreMesh(
        core_axis_name="c", subcore_axis_name="s", num_cores=NC, num_subcores=NS
    ),
    compiler_params=pltpu.CompilerParams(needs_layout_passes=False),
    scratch_types=[
        pltpu.VMEM((H,), jnp.float32),
        pltpu.VMEM((CH * F,), jnp.float32),
        pltpu.VMEM((CH * F,), jnp.float32),
        pltpu.VMEM((CH * F,), jnp.float32),
        pltpu.VMEM((CROW,), jnp.float32),
        pltpu.VMEM((2 * PB,), jnp.float32),
        pltpu.VMEM((PB,), jnp.float32),
    ],
)(_sc_body)


_TBN = 4096


def _transpose_body(in_ref, out_ref):
    out_ref[...] = in_ref[...].T


_tc_transpose = pl.pallas_call(
    _transpose_body,
    out_shape=jax.ShapeDtypeStruct((N_PTS, NL), jnp.float32),
    grid=(N_PTS // _TBN,),
    in_specs=[pl.BlockSpec((NL, _TBN), lambda i: (0, i))],
    out_specs=pl.BlockSpec((_TBN, NL), lambda i: (i, 0)),
)


def kernel(x, t, tables):
    idx = t * (TIME_RES - 1)
    i1 = jnp.floor(idx).astype(jnp.int32)
    i2 = jnp.ceil(idx).astype(jnp.int32)
    same = i1 == i2
    w1 = jnp.where(same, jnp.float32(1.0), i2.astype(jnp.float32) - idx)
    w2 = jnp.where(same, jnp.float32(0.0), idx - i1.astype(jnp.float32))
    Tm = [i / (NB - 1) for i in range(NB)]
    bs = []
    for j in range(NB):
        b = jnp.float32(1.0)
        for m in range(NB):
            if m != j:
                b = b * (t - Tm[m]) / (Tm[j] - Tm[m])
        bs.append(b)
    b = jnp.stack(bs)

    pat1 = jnp.tile(w1 * b, F)
    pat2 = jnp.tile(w2 * b, F)
    lvl = jnp.arange(NW, dtype=jnp.int32) % NL
    scal = jnp.asarray(SCALES)[lvl]
    const_rows = jnp.concatenate(
        [
            jnp.broadcast_to(pat1, (NW, LANES)),
            jnp.broadcast_to(pat2, (NW, LANES)),
            jnp.broadcast_to(scal[:, None], (NW, LANES)),
            jnp.zeros((NW, CROW - 3 * LANES), jnp.float32),
        ],
        axis=1,
    ).reshape(-1)

    slab1 = jnp.take(tables, i1, axis=0).reshape(-1)
    slab2 = jnp.take(tables, i2, axis=0).reshape(-1)
    xflat = x.reshape(-1)

    out_flat = _sc_call(slab1, slab2, const_rows, xflat)
    return _tc_transpose(out_flat.reshape(NL, N_PTS))


# speedup vs baseline: 1.2826x; 1.2826x over previous
import functools

import jax
import jax.numpy as jnp
import numpy as np
from jax import lax
from jax.experimental import pallas as pl
from jax.experimental.pallas import tpu as pltpu
from jax.experimental.pallas import tpu_sc as plsc

TIME_RES = 8
NL = 8
F = 4
H = 1 << 14
NB = 4
N_PTS = 1048576
_PLS = float(np.exp2(np.log2(32768 / 512) / (NL - 1)))
SCALES = np.array(
    [np.exp2(l * np.log2(_PLS)) * 512 - 1.0 for l in range(NL)], dtype=np.float32
)
PRIME1 = np.uint32(2654435761)
HMASK = np.uint32(H - 1)

NC = 2
NS = 16
LANES = 16
NW = NC * NS

NCHUNK = NW // NL
CHUNK_PTS = N_PTS // NCHUNK
PB = 8192
NKB = CHUNK_PTS // PB
CH = 2048
CROW = 64


def _sc_body(s1_hbm, s2_hbm, const_hbm, x_hbm, out_hbm,
             comb_v, s1buf, s2buf, pbuf, cbuf, xybuf, obuf):
    cid = lax.axis_index("c")
    sid = lax.axis_index("s")
    wid = sid * NC + cid
    level = wid % NL
    chunk = wid // NL

    pltpu.sync_copy(const_hbm.at[pl.ds(wid * CROW, CROW)], cbuf)
    pat1 = cbuf[pl.ds(0, LANES)]
    pat2 = cbuf[pl.ds(LANES, LANES)]
    scale = cbuf[pl.ds(2 * LANES, LANES)]
    iota = lax.iota(jnp.int32, LANES)

    lhf = level * (H * F)

    def stage_a(ci, carry):
        off = lhf + ci * (CH * F)
        pltpu.sync_copy(s1_hbm.at[pl.ds(off, CH * F)], s1buf)
        pltpu.sync_copy(s2_hbm.at[pl.ds(off, CH * F)], s2buf)

        def premul(g, c_):
            s = pl.ds(g * LANES, LANES)
            pbuf[s] = s1buf[s] * pat1 + s2buf[s] * pat2
            return c_

        lax.fori_loop(0, CH * F // LANES, premul, carry)

        def reduce4(g, c_):
            idx = g * (LANES * F) + iota * F
            acc = plsc.load_gather(pbuf, [idx])
            acc = acc + plsc.load_gather(pbuf, [idx + 1])
            acc = acc + plsc.load_gather(pbuf, [idx + 2])
            acc = acc + plsc.load_gather(pbuf, [idx + 3])
            comb_v[pl.ds(ci * CH + g * LANES, LANES)] = acc
            return c_

        return lax.fori_loop(0, CH // LANES, reduce4, carry)

    lax.fori_loop(0, H // CH, stage_a, 0)

    pbase = chunk * CHUNK_PTS

    def stage_b(k, carry):
        xoff = pbase + k * PB
        pltpu.sync_copy(x_hbm.at[pl.ds(2 * xoff, 2 * PB)], xybuf)

        def inner(g, c_):
            xs = plsc.load_gather(xybuf, [g * (2 * LANES) + iota * 2])
            ys = plsc.load_gather(xybuf, [g * (2 * LANES) + iota * 2 + 1])
            px = xs * scale + 0.5
            py = ys * scale + 0.5
            ix = px.astype(jnp.int32)
            iy = py.astype(jnp.int32)
            wx = px - ix.astype(jnp.float32)
            wy = py - iy.astype(jnp.float32)
            ux = ix.astype(jnp.uint32)
            uy = iy.astype(jnp.uint32)
            hy0 = uy * PRIME1
            hy1 = hy0 + PRIME1
            ux1 = ux + np.uint32(1)
            h00 = ((ux ^ hy0) & HMASK).astype(jnp.int32)
            h10 = ((ux1 ^ hy0) & HMASK).astype(jnp.int32)
            h01 = ((ux ^ hy1) & HMASK).astype(jnp.int32)
            h11 = ((ux1 ^ hy1) & HMASK).astype(jnp.int32)
            g00 = plsc.load_gather(comb_v, [h00])
            g10 = plsc.load_gather(comb_v, [h10])
            g01 = plsc.load_gather(comb_v, [h01])
            g11 = plsc.load_gather(comb_v, [h11])
            gx0 = g00 + (g10 - g00) * wx
            gx1 = g01 + (g11 - g01) * wx
            res = gx0 + (gx1 - gx0) * wy
            obuf[pl.ds(g * LANES, LANES)] = res
            return c_

        lax.fori_loop(0, PB // LANES, inner, carry)
        pltpu.sync_copy(obuf, out_hbm.at[pl.ds(level * N_PTS + xoff, PB)])
        return carry

    lax.fori_loop(0, NKB, stage_b, 0)


_sc_call = functools.partial(
    pl.kernel,
    out_type=jax.ShapeDtypeStruct((NL * N_PTS,), jnp.float32),
    mesh=plsc.VectorSubcoreMesh(
        core_axis_name="c", subcore_axis_name="s", num_cores=NC, num_subcores=NS
    ),
    compiler_params=pltpu.CompilerParams(needs_layout_passes=False),
    scratch_types=[
        pltpu.VMEM((H,), jnp.float32),
        pltpu.VMEM((CH * F,), jnp.float32),
        pltpu.VMEM((CH * F,), jnp.float32),
        pltpu.VMEM((CH * F,), jnp.float32),
        pltpu.VMEM((CROW,), jnp.float32),
        pltpu.VMEM((2 * PB,), jnp.float32),
        pltpu.VMEM((PB,), jnp.float32),
    ],
)(_sc_body)


_TBN = 4096


def _transpose_body(*refs):
    out_ref = refs[-1]
    cols = [refs[l][...] for l in range(NL)]
    out_ref[...] = jnp.stack(cols, axis=0).T


_tc_transpose = pl.pallas_call(
    _transpose_body,
    out_shape=jax.ShapeDtypeStruct((N_PTS, NL), jnp.float32),
    grid=(N_PTS // _TBN,),
    in_specs=[
        pl.BlockSpec((_TBN,), functools.partial(lambda l, i: (l * (N_PTS // _TBN) + i,), l))
        for l in range(NL)
    ],
    out_specs=pl.BlockSpec((_TBN, NL), lambda i: (i, 0)),
)


def kernel(x, t, tables):
    idx = t * (TIME_RES - 1)
    i1 = jnp.floor(idx).astype(jnp.int32)
    i2 = jnp.ceil(idx).astype(jnp.int32)
    same = i1 == i2
    w1 = jnp.where(same, jnp.float32(1.0), i2.astype(jnp.float32) - idx)
    w2 = jnp.where(same, jnp.float32(0.0), idx - i1.astype(jnp.float32))
    Tm = [i / (NB - 1) for i in range(NB)]
    bs = []
    for j in range(NB):
        b = jnp.float32(1.0)
        for m in range(NB):
            if m != j:
                b = b * (t - Tm[m]) / (Tm[j] - Tm[m])
        bs.append(b)
    b = jnp.stack(bs)

    pat1 = jnp.tile(w1 * b, F)
    pat2 = jnp.tile(w2 * b, F)
    lvl = jnp.arange(NW, dtype=jnp.int32) % NL
    scal = jnp.asarray(SCALES)[lvl]
    const_rows = jnp.concatenate(
        [
            jnp.broadcast_to(pat1, (NW, LANES)),
            jnp.broadcast_to(pat2, (NW, LANES)),
            jnp.broadcast_to(scal[:, None], (NW, LANES)),
            jnp.zeros((NW, CROW - 3 * LANES), jnp.float32),
        ],
        axis=1,
    ).reshape(-1)

    slab1 = jnp.take(tables, i1, axis=0).reshape(-1)
    slab2 = jnp.take(tables, i2, axis=0).reshape(-1)
    xflat = x.reshape(-1)

    out_flat = _sc_call(slab1, slab2, const_rows, xflat)
    return _tc_transpose(*([out_flat] * NL))


# speedup vs baseline: 6.4808x; 5.0526x over previous
import functools

import jax
import jax.numpy as jnp
import numpy as np
from jax import lax
from jax.experimental import pallas as pl
from jax.experimental.pallas import tpu as pltpu
from jax.experimental.pallas import tpu_sc as plsc

TIME_RES = 8
NL = 8
F = 4
H = 1 << 14
NB = 4
N_PTS = 1048576
_PLS = float(np.exp2(np.log2(32768 / 512) / (NL - 1)))
SCALES = np.array(
    [np.exp2(l * np.log2(_PLS)) * 512 - 1.0 for l in range(NL)], dtype=np.float32
)
PRIME1 = np.uint32(2654435761)
HMASK = np.uint32(H - 1)

NC = 2
NS = 16
LANES = 16
NW = NC * NS

NCHUNK = NW // NL
CHUNK_PTS = N_PTS // NCHUNK
PB = 8192
NKB = CHUNK_PTS // PB
HC = H // 128
CHC = 32
AW = CHC * F * 128
CROW = 160
LPW = F * 128


def _sc_body(s1_hbm, s2_hbm, const_hbm, x_hbm, out_hbm,
             comb_v, s1buf, s2buf, cbuf, xy0, xy1, obuf, sem0, sem1):
    cid = lax.axis_index("c")
    sid = lax.axis_index("s")
    wid = sid * NC + cid
    level = wid % NL
    chunk = wid // NL

    pltpu.sync_copy(const_hbm.at[pl.ds(wid * CROW, CROW)], cbuf)
    c1s = [cbuf[pl.ds(f * LANES, LANES)] for f in range(F)]
    c2s = [cbuf[pl.ds((F + f) * LANES, LANES)] for f in range(F)]
    scale = cbuf[pl.ds(2 * F * LANES, LANES)]
    iota = lax.iota(jnp.int32, LANES)

    lvl_off = level * (H * F)

    def stage_a(ci, carry):
        off = lvl_off + ci * AW
        pltpu.sync_copy(s1_hbm.at[pl.ds(off, AW)], s1buf)
        pltpu.sync_copy(s2_hbm.at[pl.ds(off, AW)], s2buf)

        def inner(g, c_):
            hcl = g // 8
            q = g % 8
            base = hcl * LPW + q * LANES
            acc = s1buf[pl.ds(base, LANES)] * c1s[0]
            acc = acc + s1buf[pl.ds(base + 128, LANES)] * c1s[1]
            acc = acc + s1buf[pl.ds(base + 256, LANES)] * c1s[2]
            acc = acc + s1buf[pl.ds(base + 384, LANES)] * c1s[3]
            acc = acc + s2buf[pl.ds(base, LANES)] * c2s[0]
            acc = acc + s2buf[pl.ds(base + 128, LANES)] * c2s[1]
            acc = acc + s2buf[pl.ds(base + 256, LANES)] * c2s[2]
            acc = acc + s2buf[pl.ds(base + 384, LANES)] * c2s[3]
            comb_v[pl.ds(ci * (CHC * 128) + hcl * 128 + q * LANES, LANES)] = acc
            return c_

        return lax.fori_loop(0, CHC * 8, inner, carry)

    lax.fori_loop(0, HC // CHC, stage_a, 0)

    pbase = chunk * CHUNK_PTS

    def dma_x(k, buf, sem):
        off = pl.multiple_of(2 * (pbase + k * PB), 2 * PB)
        pltpu.async_copy(x_hbm.at[pl.ds(off, 2 * PB)], buf, sem)

    def wait_x(buf, sem):
        pltpu.make_async_copy(x_hbm.at[pl.ds(0, 2 * PB)], buf, sem).wait()

    def process(k, buf):
        def inner(g, c_):
            base = (g // 8) * 256 + (g % 8) * LANES
            xs = buf[pl.ds(base, LANES)]
            ys = buf[pl.ds(base + 128, LANES)]
            px = xs * scale + 0.5
            py = ys * scale + 0.5
            ix = px.astype(jnp.int32)
            iy = py.astype(jnp.int32)
            wx = px - ix.astype(jnp.float32)
            wy = py - iy.astype(jnp.float32)
            ux = ix.astype(jnp.uint32)
            uy = iy.astype(jnp.uint32)
            hy0 = uy * PRIME1
            hy1 = hy0 + PRIME1
            ux1 = ux + np.uint32(1)
            h00 = ((ux ^ hy0) & HMASK).astype(jnp.int32)
            h10 = ((ux1 ^ hy0) & HMASK).astype(jnp.int32)
            h01 = ((ux ^ hy1) & HMASK).astype(jnp.int32)
            h11 = ((ux1 ^ hy1) & HMASK).astype(jnp.int32)
            g00 = plsc.load_gather(comb_v, [h00])
            g10 = plsc.load_gather(comb_v, [h10])
            g01 = plsc.load_gather(comb_v, [h01])
            g11 = plsc.load_gather(comb_v, [h11])
            gx0 = g00 + (g10 - g00) * wx
            gx1 = g01 + (g11 - g01) * wx
            res = gx0 + (gx1 - gx0) * wy
            obuf[g // 8, 0, pl.ds((g % 8) * LANES, LANES)] = res
            return c_

        lax.fori_loop(0, PB // LANES, inner, 0)
        xoff = pbase + k * PB
        row0 = pl.multiple_of(xoff // 128, 64)
        pltpu.sync_copy(
            obuf,
            out_hbm.at[pl.ds(row0, PB // 128), pl.ds(level, 1), :],
        )

    dma_x(0, xy0, sem0)

    def outer(kk, carry):
        k0 = 2 * kk
        dma_x(k0 + 1, xy1, sem1)
        wait_x(xy0, sem0)
        process(k0, xy0)

        @pl.when(k0 + 2 < NKB)
        def _():
            dma_x(k0 + 2, xy0, sem0)

        wait_x(xy1, sem1)
        process(k0 + 1, xy1)
        return carry

    lax.fori_loop(0, NKB // 2, outer, 0)


_sc_call = functools.partial(
    pl.kernel,
    out_type=jax.ShapeDtypeStruct((N_PTS // 128, NL, 128), jnp.float32),
    mesh=plsc.VectorSubcoreMesh(
        core_axis_name="c", subcore_axis_name="s", num_cores=NC, num_subcores=NS
    ),
    compiler_params=pltpu.CompilerParams(
        needs_layout_passes=False, use_tc_tiling_on_sc=False
    ),
    scratch_types=[
        pltpu.VMEM((H,), jnp.float32),
        pltpu.VMEM((AW,), jnp.float32),
        pltpu.VMEM((AW,), jnp.float32),
        pltpu.VMEM((CROW,), jnp.float32),
        pltpu.VMEM((2 * PB,), jnp.float32),
        pltpu.VMEM((2 * PB,), jnp.float32),
        pltpu.VMEM((PB // 128, 1, 128), jnp.float32),
        pltpu.SemaphoreType.DMA,
        pltpu.SemaphoreType.DMA,
    ],
)(_sc_body)


def kernel(x, t, tables):
    idx = t * (TIME_RES - 1)
    i1 = jnp.floor(idx).astype(jnp.int32)
    i2 = jnp.ceil(idx).astype(jnp.int32)
    same = i1 == i2
    w1 = jnp.where(same, jnp.float32(1.0), i2.astype(jnp.float32) - idx)
    w2 = jnp.where(same, jnp.float32(0.0), idx - i1.astype(jnp.float32))
    Tm = [i / (NB - 1) for i in range(NB)]
    bs = []
    for j in range(NB):
        b = jnp.float32(1.0)
        for m in range(NB):
            if m != j:
                b = b * (t - Tm[m]) / (Tm[j] - Tm[m])
        bs.append(b)
    b = jnp.stack(bs)

    c1 = w1 * b
    c2 = w2 * b
    lvl = jnp.arange(NW, dtype=jnp.int32) % NL
    scal = jnp.asarray(SCALES)[lvl]
    const_rows = jnp.concatenate(
        [
            jnp.broadcast_to(jnp.repeat(c1, LANES), (NW, 4 * LANES)),
            jnp.broadcast_to(jnp.repeat(c2, LANES), (NW, 4 * LANES)),
            jnp.broadcast_to(scal[:, None], (NW, LANES)),
            jnp.zeros((NW, CROW - 9 * LANES), jnp.float32),
        ],
        axis=1,
    ).reshape(-1)

    tv = tables.reshape(TIME_RES, NL, HC, 128, F).transpose(0, 1, 2, 4, 3)
    slab1 = jnp.take(tv, i1, axis=0).reshape(-1)
    slab2 = jnp.take(tv, i2, axis=0).reshape(-1)

    xv = x.reshape(N_PTS // 128, 128, 2).transpose(0, 2, 1).reshape(-1)
    out3 = _sc_call(slab1, slab2, const_rows, xv)
    return out3.transpose(0, 2, 1).reshape(N_PTS, NL)


# speedup vs baseline: 6.5857x; 1.0162x over previous
import functools

import jax
import jax.numpy as jnp
import numpy as np
from jax import lax
from jax.experimental import pallas as pl
from jax.experimental.pallas import tpu as pltpu
from jax.experimental.pallas import tpu_sc as plsc

TIME_RES = 8
NL = 8
F = 4
H = 1 << 14
NB = 4
N_PTS = 1048576
_PLS = float(np.exp2(np.log2(32768 / 512) / (NL - 1)))
SCALES = np.array(
    [np.exp2(l * np.log2(_PLS)) * 512 - 1.0 for l in range(NL)], dtype=np.float32
)
PRIME1 = np.uint32(2654435761)
HMASK = np.uint32(H - 1)

NC = 2
NS = 16
LANES = 16
NW = NC * NS

NCHUNK = NW // NL
CHUNK_PTS = N_PTS // NCHUNK
PB = 8192
NKB = CHUNK_PTS // PB
HC = H // 128
CHC = 32
AW = CHC * F * 128
CROW = 160
LPW = F * 128


def _sc_body(s1_hbm, s2_hbm, const_hbm, x_hbm, out_hbm,
             comb_v, s1buf, s2buf, cbuf, xy0, xy1, ob0, ob1,
             sem0, sem1, osem0, osem1):
    cid = lax.axis_index("c")
    sid = lax.axis_index("s")
    wid = sid * NC + cid
    level = wid % NL
    chunk = wid // NL

    pltpu.sync_copy(const_hbm.at[pl.ds(wid * CROW, CROW)], cbuf)
    c1s = [cbuf[pl.ds(f * LANES, LANES)] for f in range(F)]
    c2s = [cbuf[pl.ds((F + f) * LANES, LANES)] for f in range(F)]
    scale = cbuf[pl.ds(2 * F * LANES, LANES)]
    iota = lax.iota(jnp.int32, LANES)

    lvl_off = level * (H * F)

    def stage_a(ci, carry):
        off = lvl_off + ci * AW
        pltpu.sync_copy(s1_hbm.at[pl.ds(off, AW)], s1buf)
        pltpu.sync_copy(s2_hbm.at[pl.ds(off, AW)], s2buf)

        def inner(hcl, c_):
            base0 = hcl * LPW
            cbase = ci * (CHC * 128) + hcl * 128
            for q in range(8):
                base = base0 + q * LANES
                acc = s1buf[pl.ds(base, LANES)] * c1s[0]
                acc = acc + s1buf[pl.ds(base + 128, LANES)] * c1s[1]
                acc = acc + s1buf[pl.ds(base + 256, LANES)] * c1s[2]
                acc = acc + s1buf[pl.ds(base + 384, LANES)] * c1s[3]
                acc = acc + s2buf[pl.ds(base, LANES)] * c2s[0]
                acc = acc + s2buf[pl.ds(base + 128, LANES)] * c2s[1]
                acc = acc + s2buf[pl.ds(base + 256, LANES)] * c2s[2]
                acc = acc + s2buf[pl.ds(base + 384, LANES)] * c2s[3]
                comb_v[pl.ds(cbase + q * LANES, LANES)] = acc
            return c_

        return lax.fori_loop(0, CHC, inner, carry)

    lax.fori_loop(0, HC // CHC, stage_a, 0)

    pbase = chunk * CHUNK_PTS

    def dma_x(k, buf, sem):
        off = pl.multiple_of(2 * (pbase + k * PB), 2 * PB)
        pltpu.async_copy(x_hbm.at[pl.ds(off, 2 * PB)], buf, sem)

    def wait_x(buf, sem):
        pltpu.make_async_copy(x_hbm.at[pl.ds(0, 2 * PB)], buf, sem).wait()

    def fire_out(k, ob, osem):
        xoff = pbase + k * PB
        row0 = pl.multiple_of(xoff // 128, 64)
        pltpu.async_copy(
            ob, out_hbm.at[pl.ds(row0, PB // 128), pl.ds(level, 1), :], osem
        )

    def wait_out(ob, osem):
        pltpu.make_async_copy(
            ob, out_hbm.at[pl.ds(0, PB // 128), pl.ds(0, 1), :], osem
        ).wait()

    def process(k, buf, ob):
        def inner(c, c_):
            base0 = c * 256
            for q in range(8):
                base = base0 + q * LANES
                xs = buf[pl.ds(base, LANES)]
                ys = buf[pl.ds(base + 128, LANES)]
                px = xs * scale + 0.5
                py = ys * scale + 0.5
                ix = px.astype(jnp.int32)
                iy = py.astype(jnp.int32)
                wx = px - ix.astype(jnp.float32)
                wy = py - iy.astype(jnp.float32)
                ux = ix.astype(jnp.uint32)
                uy = iy.astype(jnp.uint32)
                hy0 = uy * PRIME1
                hy1 = hy0 + PRIME1
                ux1 = ux + np.uint32(1)
                h00 = ((ux ^ hy0) & HMASK).astype(jnp.int32)
                h10 = ((ux1 ^ hy0) & HMASK).astype(jnp.int32)
                h01 = ((ux ^ hy1) & HMASK).astype(jnp.int32)
                h11 = ((ux1 ^ hy1) & HMASK).astype(jnp.int32)
                g00 = plsc.load_gather(comb_v, [h00])
                g10 = plsc.load_gather(comb_v, [h10])
                g01 = plsc.load_gather(comb_v, [h01])
                g11 = plsc.load_gather(comb_v, [h11])
                gx0 = g00 + (g10 - g00) * wx
                gx1 = g01 + (g11 - g01) * wx
                res = gx0 + (gx1 - gx0) * wy
                ob[c, 0, pl.ds(q * LANES, LANES)] = res
            return c_

        lax.fori_loop(0, PB // 128, inner, 0)

    dma_x(0, xy0, sem0)

    def outer(kk, carry):
        k0 = 2 * kk
        dma_x(k0 + 1, xy1, sem1)
        wait_x(xy0, sem0)

        @pl.when(kk >= 1)
        def _():
            wait_out(ob0, osem0)

        process(k0, xy0, ob0)
        fire_out(k0, ob0, osem0)

        @pl.when(k0 + 2 < NKB)
        def _():
            dma_x(k0 + 2, xy0, sem0)

        wait_x(xy1, sem1)

        @pl.when(kk >= 1)
        def _():
            wait_out(ob1, osem1)

        process(k0 + 1, xy1, ob1)
        fire_out(k0 + 1, ob1, osem1)
        return carry

    lax.fori_loop(0, NKB // 2, outer, 0)
    wait_out(ob0, osem0)
    wait_out(ob1, osem1)


_sc_call = functools.partial(
    pl.kernel,
    out_type=jax.ShapeDtypeStruct((N_PTS // 128, NL, 128), jnp.float32),
    mesh=plsc.VectorSubcoreMesh(
        core_axis_name="c", subcore_axis_name="s", num_cores=NC, num_subcores=NS
    ),
    compiler_params=pltpu.CompilerParams(
        needs_layout_passes=False, use_tc_tiling_on_sc=False
    ),
    scratch_types=[
        pltpu.VMEM((H,), jnp.float32),
        pltpu.VMEM((AW,), jnp.float32),
        pltpu.VMEM((AW,), jnp.float32),
        pltpu.VMEM((CROW,), jnp.float32),
        pltpu.VMEM((2 * PB,), jnp.float32),
        pltpu.VMEM((2 * PB,), jnp.float32),
        pltpu.VMEM((PB // 128, 1, 128), jnp.float32),
        pltpu.VMEM((PB // 128, 1, 128), jnp.float32),
        pltpu.SemaphoreType.DMA,
        pltpu.SemaphoreType.DMA,
        pltpu.SemaphoreType.DMA,
        pltpu.SemaphoreType.DMA,
    ],
)(_sc_body)


def kernel(x, t, tables):
    idx = t * (TIME_RES - 1)
    i1 = jnp.floor(idx).astype(jnp.int32)
    i2 = jnp.ceil(idx).astype(jnp.int32)
    same = i1 == i2
    w1 = jnp.where(same, jnp.float32(1.0), i2.astype(jnp.float32) - idx)
    w2 = jnp.where(same, jnp.float32(0.0), idx - i1.astype(jnp.float32))
    Tm = [i / (NB - 1) for i in range(NB)]
    bs = []
    for j in range(NB):
        b = jnp.float32(1.0)
        for m in range(NB):
            if m != j:
                b = b * (t - Tm[m]) / (Tm[j] - Tm[m])
        bs.append(b)
    b = jnp.stack(bs)

    c1 = w1 * b
    c2 = w2 * b
    lvl = jnp.arange(NW, dtype=jnp.int32) % NL
    scal = jnp.asarray(SCALES)[lvl]
    const_rows = jnp.concatenate(
        [
            jnp.broadcast_to(jnp.repeat(c1, LANES), (NW, 4 * LANES)),
            jnp.broadcast_to(jnp.repeat(c2, LANES), (NW, 4 * LANES)),
            jnp.broadcast_to(scal[:, None], (NW, LANES)),
            jnp.zeros((NW, CROW - 9 * LANES), jnp.float32),
        ],
        axis=1,
    ).reshape(-1)

    tv = tables.reshape(TIME_RES, NL, HC, 128, F).transpose(0, 1, 2, 4, 3)
    slab1 = jnp.take(tv, i1, axis=0).reshape(-1)
    slab2 = jnp.take(tv, i2, axis=0).reshape(-1)

    xv = x.reshape(N_PTS // 128, 128, 2).transpose(0, 2, 1).reshape(-1)
    out3 = _sc_call(slab1, slab2, const_rows, xv)
    return out3.transpose(0, 2, 1).reshape(N_PTS, NL)


# speedup vs baseline: 34.6379x; 5.2595x over previous
import functools

import jax
import jax.numpy as jnp
import numpy as np
from jax import lax
from jax.experimental import pallas as pl
from jax.experimental.pallas import tpu as pltpu
from jax.experimental.pallas import tpu_sc as plsc

TIME_RES = 8
NL = 8
F = 4
H = 1 << 14
NB = 4
N_PTS = 1048576
_PLS = float(np.exp2(np.log2(32768 / 512) / (NL - 1)))
SCALES = np.array(
    [np.exp2(l * np.log2(_PLS)) * 512 - 1.0 for l in range(NL)], dtype=np.float32
)
PRIME1 = np.uint32(2654435761)
HMASK = np.uint32(H - 1)

NC = 2
NS = 16
LANES = 16
NW = NC * NS

NCHUNK = NW // NL
CHUNK_PTS = N_PTS // NCHUNK
PB = 8192
NKB = CHUNK_PTS // PB
HC = H // 128
CHC = 32
AW = CHC * F * 128
CROW = 160
LPW = F * 128


def _sc_body(s1_hbm, s2_hbm, const_hbm, x_hbm, out_hbm,
             comb_v, s1buf, s2buf, cbuf, xy0, xy1, ob0, ob1,
             sem0, sem1, osem0, osem1):
    cid = lax.axis_index("c")
    sid = lax.axis_index("s")
    wid = sid * NC + cid
    level = wid % NL
    chunk = wid // NL

    pltpu.sync_copy(const_hbm.at[pl.ds(wid * CROW, CROW)], cbuf)
    c1s = [cbuf[pl.ds(f * LANES, LANES)] for f in range(F)]
    c2s = [cbuf[pl.ds((F + f) * LANES, LANES)] for f in range(F)]
    scale = cbuf[pl.ds(2 * F * LANES, LANES)]
    iota = lax.iota(jnp.int32, LANES)

    lvl_off = level * (H * F)

    def stage_a(ci, carry):
        off = lvl_off + ci * AW
        pltpu.sync_copy(s1_hbm.at[pl.ds(off, AW)], s1buf)
        pltpu.sync_copy(s2_hbm.at[pl.ds(off, AW)], s2buf)

        def inner(hcl, c_):
            base0 = hcl * LPW
            cbase = ci * (CHC * 128) + hcl * 128
            for q in range(8):
                base = base0 + q * LANES
                acc = s1buf[pl.ds(base, LANES)] * c1s[0]
                acc = acc + s1buf[pl.ds(base + 128, LANES)] * c1s[1]
                acc = acc + s1buf[pl.ds(base + 256, LANES)] * c1s[2]
                acc = acc + s1buf[pl.ds(base + 384, LANES)] * c1s[3]
                acc = acc + s2buf[pl.ds(base, LANES)] * c2s[0]
                acc = acc + s2buf[pl.ds(base + 128, LANES)] * c2s[1]
                acc = acc + s2buf[pl.ds(base + 256, LANES)] * c2s[2]
                acc = acc + s2buf[pl.ds(base + 384, LANES)] * c2s[3]
                comb_v[pl.ds(cbase + q * LANES, LANES)] = acc
            return c_

        return lax.fori_loop(0, CHC, inner, carry)

    lax.fori_loop(0, HC // CHC, stage_a, 0)

    pbase = chunk * CHUNK_PTS

    def dma_x(k, buf, sem):
        off = pl.multiple_of(2 * (pbase + k * PB), 2 * PB)
        pltpu.async_copy(x_hbm.at[pl.ds(off, 2 * PB)], buf, sem)

    def wait_x(buf, sem):
        pltpu.make_async_copy(x_hbm.at[pl.ds(0, 2 * PB)], buf, sem).wait()

    def fire_out(k, ob, osem):
        xoff = pbase + k * PB
        row0 = pl.multiple_of(xoff // 128, 64)
        pltpu.async_copy(
            ob, out_hbm.at[pl.ds(row0, PB // 128), pl.ds(level, 1), :], osem
        )

    def wait_out(ob, osem):
        pltpu.make_async_copy(
            ob, out_hbm.at[pl.ds(0, PB // 128), pl.ds(0, 1), :], osem
        ).wait()

    def process(k, buf, ob):
        @functools.partial(plsc.parallel_loop, 0, PB // 128, unroll=2)
        def inner(c):
            base0 = c * 256
            for q in range(8):
                base = base0 + q * LANES
                xs = buf[pl.ds(base, LANES)]
                ys = buf[pl.ds(base + 128, LANES)]
                px = xs * scale + 0.5
                py = ys * scale + 0.5
                ix = px.astype(jnp.int32)
                iy = py.astype(jnp.int32)
                wx = px - ix.astype(jnp.float32)
                wy = py - iy.astype(jnp.float32)
                ux = ix.astype(jnp.uint32)
                uy = iy.astype(jnp.uint32)
                hy0 = uy * PRIME1
                hy1 = hy0 + PRIME1
                ux1 = ux + np.uint32(1)
                h00 = ((ux ^ hy0) & HMASK).astype(jnp.int32)
                h10 = ((ux1 ^ hy0) & HMASK).astype(jnp.int32)
                h01 = ((ux ^ hy1) & HMASK).astype(jnp.int32)
                h11 = ((ux1 ^ hy1) & HMASK).astype(jnp.int32)
                g00 = plsc.load_gather(comb_v, [h00])
                g10 = plsc.load_gather(comb_v, [h10])
                g01 = plsc.load_gather(comb_v, [h01])
                g11 = plsc.load_gather(comb_v, [h11])
                gx0 = g00 + (g10 - g00) * wx
                gx1 = g01 + (g11 - g01) * wx
                res = gx0 + (gx1 - gx0) * wy
                ob[c, 0, pl.ds(q * LANES, LANES)] = res

    dma_x(0, xy0, sem0)

    def outer(kk, carry):
        k0 = 2 * kk
        dma_x(k0 + 1, xy1, sem1)
        wait_x(xy0, sem0)

        @pl.when(kk >= 1)
        def _():
            wait_out(ob0, osem0)

        process(k0, xy0, ob0)
        fire_out(k0, ob0, osem0)

        @pl.when(k0 + 2 < NKB)
        def _():
            dma_x(k0 + 2, xy0, sem0)

        wait_x(xy1, sem1)

        @pl.when(kk >= 1)
        def _():
            wait_out(ob1, osem1)

        process(k0 + 1, xy1, ob1)
        fire_out(k0 + 1, ob1, osem1)
        return carry

    lax.fori_loop(0, NKB // 2, outer, 0)
    wait_out(ob0, osem0)
    wait_out(ob1, osem1)


_sc_call = functools.partial(
    pl.kernel,
    out_type=jax.ShapeDtypeStruct((N_PTS // 128, NL, 128), jnp.float32),
    mesh=plsc.VectorSubcoreMesh(
        core_axis_name="c", subcore_axis_name="s", num_cores=NC, num_subcores=NS
    ),
    compiler_params=pltpu.CompilerParams(
        needs_layout_passes=False, use_tc_tiling_on_sc=False
    ),
    scratch_types=[
        pltpu.VMEM((H,), jnp.float32),
        pltpu.VMEM((AW,), jnp.float32),
        pltpu.VMEM((AW,), jnp.float32),
        pltpu.VMEM((CROW,), jnp.float32),
        pltpu.VMEM((2 * PB,), jnp.float32),
        pltpu.VMEM((2 * PB,), jnp.float32),
        pltpu.VMEM((PB // 128, 1, 128), jnp.float32),
        pltpu.VMEM((PB // 128, 1, 128), jnp.float32),
        pltpu.SemaphoreType.DMA,
        pltpu.SemaphoreType.DMA,
        pltpu.SemaphoreType.DMA,
        pltpu.SemaphoreType.DMA,
    ],
)(_sc_body)


def kernel(x, t, tables):
    idx = t * (TIME_RES - 1)
    i1 = jnp.floor(idx).astype(jnp.int32)
    i2 = jnp.ceil(idx).astype(jnp.int32)
    same = i1 == i2
    w1 = jnp.where(same, jnp.float32(1.0), i2.astype(jnp.float32) - idx)
    w2 = jnp.where(same, jnp.float32(0.0), idx - i1.astype(jnp.float32))
    Tm = [i / (NB - 1) for i in range(NB)]
    bs = []
    for j in range(NB):
        b = jnp.float32(1.0)
        for m in range(NB):
            if m != j:
                b = b * (t - Tm[m]) / (Tm[j] - Tm[m])
        bs.append(b)
    b = jnp.stack(bs)

    c1 = w1 * b
    c2 = w2 * b
    lvl = jnp.arange(NW, dtype=jnp.int32) % NL
    scal = jnp.asarray(SCALES)[lvl]
    const_rows = jnp.concatenate(
        [
            jnp.broadcast_to(jnp.repeat(c1, LANES), (NW, 4 * LANES)),
            jnp.broadcast_to(jnp.repeat(c2, LANES), (NW, 4 * LANES)),
            jnp.broadcast_to(scal[:, None], (NW, LANES)),
            jnp.zeros((NW, CROW - 9 * LANES), jnp.float32),
        ],
        axis=1,
    ).reshape(-1)

    tv = tables.reshape(TIME_RES, NL, HC, 128, F).transpose(0, 1, 2, 4, 3)
    slab1 = jnp.take(tv, i1, axis=0).reshape(-1)
    slab2 = jnp.take(tv, i2, axis=0).reshape(-1)

    xv = x.reshape(N_PTS // 128, 128, 2).transpose(0, 2, 1).reshape(-1)
    out3 = _sc_call(slab1, slab2, const_rows, xv)
    return out3.transpose(0, 2, 1).reshape(N_PTS, NL)
